# baseline (speedup 1.0000x reference)
import functools

import jax
import jax.numpy as jnp
import numpy as np
from jax import lax
from jax.experimental import pallas as pl
from jax.experimental.pallas import tpu as pltpu
from jax.experimental.pallas import tpu_sc as plsc

TOP_P = 0.9
NEG_INF = float("-inf")

ROWS = 32
V = 1000000
CH = 20000
NCHUNK = V // CH
NB = 4096
HWORDS = NB * 16

SUB = 7816
LANE = 128
PADDED = SUB * LANE

_SIGN = np.uint32(0x80000000)


def _key16(v, u):
    return jnp.where(v < 0.0, ~u, u ^ _SIGN)


def _sc_body(x_hbm, key_out, s_out, cbuf, hist, kscr, sscr):
    wid = lax.axis_index("c") * 16 + lax.axis_index("s")
    lane = lax.iota(jnp.int32, 16)
    zero16 = jnp.zeros((16,), jnp.float32)

    def hzero(j, _):
        hist[pl.ds(j * 16, 16)] = zero16
        return 0

    lax.fori_loop(0, NB, hzero, 0)

    row0 = wid * V

    def chunk_a(c, z):
        pltpu.sync_copy(x_hbm.at[pl.ds(row0 + c * CH, CH)], cbuf)

        def vec_a(i, zz):
            v = cbuf[pl.ds(i * 16, 16)]
            u = lax.bitcast_convert_type(v, jnp.uint32)
            key = _key16(v, u)
            p = jnp.exp(v)
            b1 = (key >> jnp.uint32(20)).astype(jnp.int32)
            plsc.addupdate_scatter(hist, [(b1 << 4) + lane], p)
            return zz + p

        return lax.fori_loop(0, CH // 16, vec_a, z)

    zvec = lax.fori_loop(0, NCHUNK, chunk_a, zero16)
    target = TOP_P * jnp.sum(zvec)

    def scan(carry_init):
        def body(i, carry):
            run, cbin, sinc, bmass = carry
            j = NB - 1 - i
            m = jnp.sum(hist[pl.ds(j * 16, 16)])
            hist[pl.ds(j * 16, 16)] = zero16
            newrun = run + m
            crossed = (run < target) & (newrun >= target)
            cbin = jnp.where(crossed, j, cbin)
            sinc = jnp.where(crossed, newrun, sinc)
            bmass = jnp.where(crossed, m, bmass)
            return (newrun, cbin, sinc, bmass)

        return lax.fori_loop(0, NB, body, carry_init)

    _, c1, s_inc_a, bm_a = scan((jnp.float32(0.0), jnp.int32(0),
                                 jnp.float32(0.0), jnp.float32(0.0)))
    g_above = s_inc_a - bm_a
    c1u = c1.astype(jnp.uint32)

    def chunk_b(c, _):
        pltpu.sync_copy(x_hbm.at[pl.ds(row0 + c * CH, CH)], cbuf)

        def vec_b(i, __):
            v = cbuf[pl.ds(i * 16, 16)]
            u = lax.bitcast_convert_type(v, jnp.uint32)
            key = _key16(v, u)
            p = jnp.exp(v)
            is_in = (key >> jnp.uint32(20)) == c1u
            b2 = ((key >> jnp.uint32(8)) & jnp.uint32(0xFFF)).astype(jnp.int32)
            plsc.addupdate_scatter(hist, [(b2 << 4) + lane], p, mask=is_in)
            return 0

        return lax.fori_loop(0, CH // 16, vec_b, 0)

    lax.fori_loop(0, NCHUNK, chunk_b, 0)

    _, c2, s_inc_b, _ = scan((g_above, jnp.int32(0), g_above,
                              jnp.float32(0.0)))
    c2u = c2.astype(jnp.uint32)

    tkey = ((c1u << jnp.uint32(12)) | c2u) << jnp.uint32(8)
    kscr[...] = jnp.full((16,), tkey, jnp.uint32)
    sscr[...] = jnp.full((16,), s_inc_b, jnp.float32)
    pltpu.sync_copy(kscr, key_out.at[pl.ds(wid * 16, 16)])
    pltpu.sync_copy(sscr, s_out.at[pl.ds(wid * 16, 16)])


_sc_threshold = functools.partial(
    pl.kernel,
    out_type=[
        jax.ShapeDtypeStruct((ROWS * 16,), jnp.uint32),
        jax.ShapeDtypeStruct((ROWS * 16,), jnp.float32),
    ],
    mesh=plsc.VectorSubcoreMesh(core_axis_name="c", subcore_axis_name="s"),
    compiler_params=pltpu.CompilerParams(needs_layout_passes=False),
    scratch_types=[
        pltpu.VMEM((CH,), jnp.float32),
        pltpu.VMEM((HWORDS,), jnp.float32),
        pltpu.VMEM((16,), jnp.uint32),
        pltpu.VMEM((16,), jnp.float32),
    ],
)(_sc_body)


def _finalize_kernel(x_ref, k_ref, s_ref, o_ref):
    x = x_ref[0]
    kth = k_ref[0, 0, 0]
    c = jnp.log(s_ref[0, 0, 0])
    u = lax.bitcast_convert_type(x, jnp.uint32)
    key = jnp.where(x < 0.0, ~u, u ^ _SIGN)
    o_ref[0] = jnp.where(key >= kth, x - c, NEG_INF)


@jax.jit
def kernel(logprobs):
    b, v = logprobs.shape
    keys, svals = _sc_threshold(logprobs.reshape(-1))
    keys = keys.reshape(b, 1, 16)
    svals = svals.reshape(b, 1, 16)
    x = jnp.pad(logprobs, ((0, 0), (0, PADDED - v)), constant_values=NEG_INF)
    x = x.reshape(b, SUB, LANE)
    out = pl.pallas_call(
        _finalize_kernel,
        grid=(b,),
        in_specs=[
            pl.BlockSpec((1, SUB, LANE), lambda i: (i, 0, 0)),
            pl.BlockSpec((1, 1, 16), lambda i: (i, 0, 0)),
            pl.BlockSpec((1, 1, 16), lambda i: (i, 0, 0)),
        ],
        out_specs=pl.BlockSpec((1, SUB, LANE), lambda i: (i, 0, 0)),
        out_shape=jax.ShapeDtypeStruct((b, SUB, LANE), jnp.float32),
    )(x, keys, svals)
    return out.reshape(b, PADDED)[:, :v]


# speedup vs baseline: 1.0212x; 1.0212x over previous
import functools

import jax
import jax.numpy as jnp
import numpy as np
from jax import lax
from jax.experimental import pallas as pl
from jax.experimental.pallas import tpu as pltpu
from jax.experimental.pallas import tpu_sc as plsc

TOP_P = 0.9
NEG_INF = float("-inf")

ROWS = 32
V = 1000000
CH = 20000
NCHUNK = V // CH
NB = 4096
HWORDS = NB * 16

SUB = 7816
LANE = 128
PADDED = SUB * LANE

_SIGN = np.uint32(0x80000000)


def _key16(v, u):
    return jnp.where(v < 0.0, ~u, u ^ _SIGN)


def _sc_body(x_hbm, key_out, s_out, cbuf, hist, kscr, sscr):
    wid = lax.axis_index("c") * 16 + lax.axis_index("s")
    lane = lax.iota(jnp.int32, 16)
    zero16 = jnp.zeros((16,), jnp.float32)

    def hzero(j, _):
        hist[pl.ds(j * 16, 16)] = zero16
        return 0

    lax.fori_loop(0, NB, hzero, 0, unroll=8)

    row0 = wid * V

    def chunk_a(c, z):
        pltpu.sync_copy(x_hbm.at[pl.ds(row0 + c * CH, CH)], cbuf)

        def vec_a(i, zz):
            v = cbuf[pl.ds(i * 16, 16)]
            u = lax.bitcast_convert_type(v, jnp.uint32)
            key = _key16(v, u)
            p = jnp.exp(v)
            b1 = (key >> jnp.uint32(20)).astype(jnp.int32)
            plsc.addupdate_scatter(hist, [(b1 << 4) + lane], p)
            return zz + p

        return lax.fori_loop(0, CH // 16, vec_a, z, unroll=8)

    zvec = lax.fori_loop(0, NCHUNK, chunk_a, zero16)
    target = TOP_P * jnp.sum(zvec)

    def scan(carry_init):
        def body(i, carry):
            run, cbin, sinc, bmass = carry
            j = NB - 1 - i
            m = jnp.sum(hist[pl.ds(j * 16, 16)])
            hist[pl.ds(j * 16, 16)] = zero16
            newrun = run + m
            crossed = (run < target) & (newrun >= target)
            cbin = jnp.where(crossed, j, cbin)
            sinc = jnp.where(crossed, newrun, sinc)
            bmass = jnp.where(crossed, m, bmass)
            return (newrun, cbin, sinc, bmass)

        return lax.fori_loop(0, NB, body, carry_init, unroll=4)

    _, c1, s_inc_a, bm_a = scan((jnp.float32(0.0), jnp.int32(0),
                                 jnp.float32(0.0), jnp.float32(0.0)))
    g_above = s_inc_a - bm_a
    c1u = c1.astype(jnp.uint32)

    def chunk_b(c, _):
        pltpu.sync_copy(x_hbm.at[pl.ds(row0 + c * CH, CH)], cbuf)

        def vec_b(i, __):
            v = cbuf[pl.ds(i * 16, 16)]
            u = lax.bitcast_convert_type(v, jnp.uint32)
            key = _key16(v, u)
            p = jnp.exp(v)
            is_in = (key >> jnp.uint32(20)) == c1u
            b2 = ((key >> jnp.uint32(8)) & jnp.uint32(0xFFF)).astype(jnp.int32)
            plsc.addupdate_scatter(hist, [(b2 << 4) + lane], p, mask=is_in)
            return 0

        return lax.fori_loop(0, CH // 16, vec_b, 0, unroll=8)

    lax.fori_loop(0, NCHUNK, chunk_b, 0)

    _, c2, s_inc_b, _ = scan((g_above, jnp.int32(0), g_above,
                              jnp.float32(0.0)))
    c2u = c2.astype(jnp.uint32)

    tkey = ((c1u << jnp.uint32(12)) | c2u) << jnp.uint32(8)
    kscr[...] = jnp.full((16,), tkey, jnp.uint32)
    sscr[...] = jnp.full((16,), s_inc_b, jnp.float32)
    pltpu.sync_copy(kscr, key_out.at[pl.ds(wid * 16, 16)])
    pltpu.sync_copy(sscr, s_out.at[pl.ds(wid * 16, 16)])


_sc_threshold = functools.partial(
    pl.kernel,
    out_type=[
        jax.ShapeDtypeStruct((ROWS * 16,), jnp.uint32),
        jax.ShapeDtypeStruct((ROWS * 16,), jnp.float32),
    ],
    mesh=plsc.VectorSubcoreMesh(core_axis_name="c", subcore_axis_name="s"),
    compiler_params=pltpu.CompilerParams(needs_layout_passes=False),
    scratch_types=[
        pltpu.VMEM((CH,), jnp.float32),
        pltpu.VMEM((HWORDS,), jnp.float32),
        pltpu.VMEM((16,), jnp.uint32),
        pltpu.VMEM((16,), jnp.float32),
    ],
)(_sc_body)


def _finalize_kernel(x_ref, k_ref, s_ref, o_ref):
    x = x_ref[0]
    kth = k_ref[0, 0, 0]
    c = jnp.log(s_ref[0, 0, 0])
    u = lax.bitcast_convert_type(x, jnp.uint32)
    key = jnp.where(x < 0.0, ~u, u ^ _SIGN)
    o_ref[0] = jnp.where(key >= kth, x - c, NEG_INF)


@jax.jit
def kernel(logprobs):
    b, v = logprobs.shape
    keys, svals = _sc_threshold(logprobs.reshape(-1))
    keys = keys.reshape(b, 1, 16)
    svals = svals.reshape(b, 1, 16)
    x = jnp.pad(logprobs, ((0, 0), (0, PADDED - v)), constant_values=NEG_INF)
    x = x.reshape(b, SUB, LANE)
    out = pl.pallas_call(
        _finalize_kernel,
        grid=(b,),
        in_specs=[
            pl.BlockSpec((1, SUB, LANE), lambda i: (i, 0, 0)),
            pl.BlockSpec((1, 1, 16), lambda i: (i, 0, 0)),
            pl.BlockSpec((1, 1, 16), lambda i: (i, 0, 0)),
        ],
        out_specs=pl.BlockSpec((1, SUB, LANE), lambda i: (i, 0, 0)),
        out_shape=jax.ShapeDtypeStruct((b, SUB, LANE), jnp.float32),
    )(x, keys, svals)
    return out.reshape(b, PADDED)[:, :v]


# speedup vs baseline: 1.3899x; 1.3611x over previous
import functools

import jax
import jax.numpy as jnp
import numpy as np
from jax import lax
from jax.experimental import pallas as pl
from jax.experimental.pallas import tpu as pltpu
from jax.experimental.pallas import tpu_sc as plsc

TOP_P = 0.9
NEG_INF = float("-inf")

ROWS = 32
V = 1000000
CH = 20000
NCHUNK = V // CH
NB = 4096
HWORDS = NB * 16

SUB = 7816
LANE = 128
PADDED = SUB * LANE

_SIGN = np.uint32(0x80000000)


def _key16(v, u):
    return jnp.where(v < 0.0, ~u, u ^ _SIGN)


def _sc_body(x_hbm, key_out, s_out, cbuf, hist, kscr, sscr, sem_a, sem_b):
    wid = lax.axis_index("c") * 16 + lax.axis_index("s")
    lane = lax.iota(jnp.int32, 16)
    zero16 = jnp.zeros((16,), jnp.float32)
    row0 = wid * V

    @plsc.parallel_loop(0, NB, unroll=8)
    def _(j):
        hist[pl.ds(j * 16, 16)] = zero16

    def src(c):
        return x_hbm.at[pl.ds(row0 + c * CH, CH)]

    def slot(k):
        return cbuf.at[pl.ds(k * CH, CH)]

    def stream_row(process_chunk, init):
        pltpu.async_copy(src(0), slot(0), sem_a)

        def pair(k, acc):
            c = 2 * k
            pltpu.make_async_copy(src(c), slot(0), sem_a).wait()
            pltpu.async_copy(src(c + 1), slot(1), sem_b)
            acc = process_chunk(0, acc)
            pltpu.make_async_copy(src(c + 1), slot(1), sem_b).wait()

            @pl.when(c + 2 < NCHUNK)
            def _():
                pltpu.async_copy(src(c + 2), slot(0), sem_a)

            return process_chunk(CH, acc)

        return lax.fori_loop(0, NCHUNK // 2, pair, init)

    def chunk_a(base, z):
        def body(i, zz):
            v = cbuf[pl.ds(base + i * 16, 16)]
            u = lax.bitcast_convert_type(v, jnp.uint32)
            key = _key16(v, u)
            p = jnp.exp(v)
            b1 = (key >> jnp.uint32(20)).astype(jnp.int32)
            plsc.addupdate_scatter(hist, [(b1 << 4) + lane], p)
            return zz + p

        return plsc.parallel_loop(0, CH // 16, unroll=8, carry=z)(body)

    zvec = stream_row(chunk_a, zero16)
    target = TOP_P * jnp.sum(zvec)

    def scan(init):
        def body(i, carry):
            run, cbin, sinc, bmass = carry
            j = NB - 1 - i
            m = jnp.sum(hist[pl.ds(j * 16, 16)])
            hist[pl.ds(j * 16, 16)] = zero16
            newrun = run + m
            crossed = (run < target) & (newrun >= target)
            cbin = jnp.where(crossed, j, cbin)
            sinc = jnp.where(crossed, newrun, sinc)
            bmass = jnp.where(crossed, m, bmass)
            return (newrun, cbin, sinc, bmass)

        return plsc.parallel_loop(0, NB, unroll=8, carry=init)(body)

    _, c1, s_inc_a, bm_a = scan((jnp.float32(0.0), jnp.int32(0),
                                 jnp.float32(0.0), jnp.float32(0.0)))
    g_above = s_inc_a - bm_a
    c1u = c1.astype(jnp.uint32)

    def chunk_b(base, acc):
        def body(i):
            v = cbuf[pl.ds(base + i * 16, 16)]
            u = lax.bitcast_convert_type(v, jnp.uint32)
            key = _key16(v, u)
            p = jnp.exp(v)
            is_in = (key >> jnp.uint32(20)) == c1u
            b2 = ((key >> jnp.uint32(8)) & jnp.uint32(0xFFF)).astype(jnp.int32)
            plsc.addupdate_scatter(hist, [(b2 << 4) + lane], p, mask=is_in)

        plsc.parallel_loop(0, CH // 16, unroll=8)(body)
        return acc

    stream_row(chunk_b, 0)

    _, c2, s_inc_b, _ = scan((g_above, jnp.int32(0), g_above,
                              jnp.float32(0.0)))
    c2u = c2.astype(jnp.uint32)

    tkey = ((c1u << jnp.uint32(12)) | c2u) << jnp.uint32(8)
    kscr[...] = jnp.full((16,), tkey, jnp.uint32)
    sscr[...] = jnp.full((16,), s_inc_b, jnp.float32)
    pltpu.sync_copy(kscr, key_out.at[pl.ds(wid * 16, 16)])
    pltpu.sync_copy(sscr, s_out.at[pl.ds(wid * 16, 16)])


_sc_threshold = functools.partial(
    pl.kernel,
    out_type=[
        jax.ShapeDtypeStruct((ROWS * 16,), jnp.uint32),
        jax.ShapeDtypeStruct((ROWS * 16,), jnp.float32),
    ],
    mesh=plsc.VectorSubcoreMesh(core_axis_name="c", subcore_axis_name="s"),
    compiler_params=pltpu.CompilerParams(needs_layout_passes=False),
    scratch_types=[
        pltpu.VMEM((2 * CH,), jnp.float32),
        pltpu.VMEM((HWORDS,), jnp.float32),
        pltpu.VMEM((16,), jnp.uint32),
        pltpu.VMEM((16,), jnp.float32),
        pltpu.SemaphoreType.DMA,
        pltpu.SemaphoreType.DMA,
    ],
)(_sc_body)


def _finalize_kernel(x_ref, k_ref, s_ref, o_ref):
    x = x_ref[0]
    kth = k_ref[0, 0, 0]
    c = jnp.log(s_ref[0, 0, 0])
    u = lax.bitcast_convert_type(x, jnp.uint32)
    key = jnp.where(x < 0.0, ~u, u ^ _SIGN)
    o_ref[0] = jnp.where(key >= kth, x - c, NEG_INF)


@jax.jit
def kernel(logprobs):
    b, v = logprobs.shape
    keys, svals = _sc_threshold(logprobs.reshape(-1))
    keys = keys.reshape(b, 1, 16)
    svals = svals.reshape(b, 1, 16)
    x = jnp.pad(logprobs, ((0, 0), (0, PADDED - v)), constant_values=NEG_INF)
    x = x.reshape(b, SUB, LANE)
    out = pl.pallas_call(
        _finalize_kernel,
        grid=(b,),
        in_specs=[
            pl.BlockSpec((1, SUB, LANE), lambda i: (i, 0, 0)),
            pl.BlockSpec((1, 1, 16), lambda i: (i, 0, 0)),
            pl.BlockSpec((1, 1, 16), lambda i: (i, 0, 0)),
        ],
        out_specs=pl.BlockSpec((1, SUB, LANE), lambda i: (i, 0, 0)),
        out_shape=jax.ShapeDtypeStruct((b, SUB, LANE), jnp.float32),
    )(x, keys, svals)
    return out.reshape(b, PADDED)[:, :v]


# speedup vs baseline: 5.8637x; 4.2187x over previous
import functools

import jax
import jax.numpy as jnp
import numpy as np
from jax import lax
from jax.experimental import pallas as pl
from jax.experimental.pallas import tpu as pltpu
from jax.experimental.pallas import tpu_sc as plsc

TOP_P = 0.9
NEG_INF = float("-inf")

ROWS = 32
V = 1000000
NB = 4096
HWORDS = NB * 16

SUB = 8192
LANE = 128
PADDED = SUB * LANE

CSUB = 128
CVREG = CSUB * LANE // 16
NCHUNK = SUB // CSUB

_SIGN = np.uint32(0x80000000)


def _key16(v, u):
    return jnp.where(v < 0.0, ~u, u ^ _SIGN)


def _sc_body(x_hbm, out_hbm, cbuf, hist, kscr, sscr, sem_a, sem_b):
    wid = lax.axis_index("c") * 16 + lax.axis_index("s")
    lane = lax.iota(jnp.int32, 16)
    zero16 = jnp.zeros((16,), jnp.float32)

    @plsc.parallel_loop(0, NB, unroll=8)
    def _(j):
        hist[pl.ds(j * 16, 16)] = zero16

    def src(c):
        return x_hbm.at[wid, pl.ds(c * CSUB, CSUB), :]

    def slot(k):
        return cbuf.at[pl.ds(k * CSUB, CSUB), :]

    def stream_row(process_chunk, init):
        pltpu.async_copy(src(0), slot(0), sem_a)

        def pair(k, acc):
            c = 2 * k
            pltpu.make_async_copy(src(c), slot(0), sem_a).wait()
            pltpu.async_copy(src(c + 1), slot(1), sem_b)
            acc = process_chunk(0, acc)
            pltpu.make_async_copy(src(c + 1), slot(1), sem_b).wait()

            @pl.when(c + 2 < NCHUNK)
            def _():
                pltpu.async_copy(src(c + 2), slot(0), sem_a)

            return process_chunk(CSUB, acc)

        return lax.fori_loop(0, NCHUNK // 2, pair, init)

    def chunk_a(base, z):
        def body(i, zz):
            v = cbuf[base + (i >> 3), pl.ds((i & 7) * 16, 16)]
            u = lax.bitcast_convert_type(v, jnp.uint32)
            key = _key16(v, u)
            p = jnp.exp(v)
            b1 = (key >> jnp.uint32(20)).astype(jnp.int32)
            plsc.addupdate_scatter(hist, [(b1 << 4) + lane], p)
            return zz + p

        return plsc.parallel_loop(0, CVREG, unroll=8, carry=z)(body)

    zvec = stream_row(chunk_a, zero16)
    target = TOP_P * jnp.sum(zvec)

    def scan(init):
        def body(i, carry):
            run, cbin, sinc, bmass = carry
            j = NB - 1 - i
            m = jnp.sum(hist[pl.ds(j * 16, 16)])
            hist[pl.ds(j * 16, 16)] = zero16
            newrun = run + m
            crossed = (run < target) & (newrun >= target)
            cbin = jnp.where(crossed, j, cbin)
            sinc = jnp.where(crossed, newrun, sinc)
            bmass = jnp.where(crossed, m, bmass)
            return (newrun, cbin, sinc, bmass)

        return plsc.parallel_loop(0, NB, unroll=8, carry=init)(body)

    _, c1, s_inc_a, bm_a = scan((jnp.float32(0.0), jnp.int32(0),
                                 jnp.float32(0.0), jnp.float32(0.0)))
    g_above = s_inc_a - bm_a
    c1u = c1.astype(jnp.uint32)

    def chunk_b(base, acc):
        def body(i):
            v = cbuf[base + (i >> 3), pl.ds((i & 7) * 16, 16)]
            u = lax.bitcast_convert_type(v, jnp.uint32)
            key = _key16(v, u)
            p = jnp.exp(v)
            is_in = (key >> jnp.uint32(20)) == c1u
            b2 = ((key >> jnp.uint32(8)) & jnp.uint32(0xFFF)).astype(jnp.int32)
            plsc.addupdate_scatter(hist, [(b2 << 4) + lane], p, mask=is_in)

        plsc.parallel_loop(0, CVREG, unroll=8)(body)
        return acc

    stream_row(chunk_b, 0)

    _, c2, s_inc_b, _ = scan((g_above, jnp.int32(0), g_above,
                              jnp.float32(0.0)))
    c2u = c2.astype(jnp.uint32)

    tkey = ((c1u << jnp.uint32(12)) | c2u) << jnp.uint32(8)
    kscr[...] = jnp.full((16,), tkey, jnp.uint32)
    sscr[...] = lax.bitcast_convert_type(
        jnp.full((16,), s_inc_b, jnp.float32), jnp.uint32)
    pltpu.sync_copy(kscr, out_hbm.at[pl.ds(wid * 32, 16)])
    pltpu.sync_copy(sscr, out_hbm.at[pl.ds(wid * 32 + 16, 16)])


_sc_threshold = functools.partial(
    pl.kernel,
    out_type=jax.ShapeDtypeStruct((ROWS * 32,), jnp.uint32),
    mesh=plsc.VectorSubcoreMesh(core_axis_name="c", subcore_axis_name="s"),
    compiler_params=pltpu.CompilerParams(needs_layout_passes=False),
    scratch_types=[
        pltpu.VMEM((2 * CSUB, LANE), jnp.float32),
        pltpu.VMEM((HWORDS,), jnp.float32),
        pltpu.VMEM((16,), jnp.uint32),
        pltpu.VMEM((16,), jnp.uint32),
        pltpu.SemaphoreType.DMA,
        pltpu.SemaphoreType.DMA,
    ],
)(_sc_body)


def _finalize_kernel(x_ref, k_ref, s_ref, o_ref):
    x = x_ref[0]
    kth = k_ref[0, 0, 0]
    c = jnp.log(s_ref[0, 0, 0])
    u = lax.bitcast_convert_type(x, jnp.uint32)
    key = jnp.where(x < 0.0, ~u, u ^ _SIGN)
    o_ref[0] = jnp.where(key >= kth, x - c, NEG_INF)


@jax.jit
def kernel(logprobs):
    b, v = logprobs.shape
    x = jnp.pad(logprobs, ((0, 0), (0, PADDED - v)), constant_values=NEG_INF)
    x = x.reshape(b, SUB, LANE)
    packed = _sc_threshold(x).reshape(b, 2, 16)
    keys = packed[:, 0:1, :]
    svals = lax.bitcast_convert_type(packed[:, 1:2, :], jnp.float32)
    out = pl.pallas_call(
        _finalize_kernel,
        grid=(b,),
        in_specs=[
            pl.BlockSpec((1, SUB, LANE), lambda i: (i, 0, 0)),
            pl.BlockSpec((1, 1, 16), lambda i: (i, 0, 0)),
            pl.BlockSpec((1, 1, 16), lambda i: (i, 0, 0)),
        ],
        out_specs=pl.BlockSpec((1, SUB, LANE), lambda i: (i, 0, 0)),
        out_shape=jax.ShapeDtypeStruct((b, SUB, LANE), jnp.float32),
    )(x, keys, svals)
    return out.reshape(b, PADDED)[:, :v]
